# baseline (speedup 1.0000x reference)
import jax
import jax.numpy as jnp
from jax import lax
from jax.experimental import pallas as pl
from jax.experimental.pallas import tpu as pltpu
from jax.experimental.pallas import tpu_sc as plsc

NU = 25000
NI = 25000
N = NU + NI
NE = 800000
D = 64
H = 32
B = 4096
NS = 16
K = 128
SE = 256
EPT = 51200
NE_PAD = EPT * NS
NST = EPT // SE
NST_TOT = NE_PAD // SE
NP_ = 50048
RPT = NP_ // NS
NRC = 25
BPT = B // NS
PIB = jax.lax.GatherScatterMode.PROMISE_IN_BOUNDS
F32 = jnp.float32
I32 = jnp.int32


def _vsplat(vec, j):
    return lax.gather(
        vec, jnp.full((16, 1), j, I32),
        dimension_numbers=lax.GatherDimensionNumbers(
            offset_dims=(), collapsed_slice_dims=(0,), start_index_map=(0,)),
        slice_sizes=(1,), mode=PIB)


def _sc_body(allemb, edata, zeros_slab, users, items,
             part, a0, a1, a2, a3,
             acc,
             ed0, ed1, ed2, dx0, dx1, dx2, rw0, rw1, rw2,
             part_v,
             se0, se1, se2, sg0, sg1, sg2, ss0, ss1, ss2):
    c = lax.axis_index("c")
    s = lax.axis_index("s")
    node_off = c * NP_
    ed = (ed0, ed1, ed2)
    dx = (dx0, dx1, dx2)
    rw = (rw0, rw1, rw2)
    semE = (se0, se1, se2)
    semG = (sg0, sg1, sg2)
    semS = (ss0, ss1, ss2)
    LO = pl.ds(0, K)
    HI = pl.ds(K, K)

    def phase0(col0):
        def it(i, carry):
            r = s * RPT + jnp.minimum(i * K, RPT - K)
            pltpu.sync_copy(allemb.at[pl.ds(r, K), pl.ds(col0, H)],
                            rw0.at[LO])
            pltpu.sync_copy(rw0.at[LO], a0.at[pl.ds(node_off + r, K)])
            return carry
        lax.fori_loop(0, NRC, it, None)

    pl.when(c == 0)(lambda: phase0(0))
    pl.when(c == 1)(lambda: phase0(H))
    pltpu.sync_copy(zeros_slab, acc.at[pl.ds(s * RPT, RPT)])
    plsc.subcore_barrier()

    def layer(src_tab, dst_tab):
        base0 = s * NST

        def issue_e(st, k):
            pltpu.async_copy(edata.at[c, base0 + st], ed[k], semE[k])

        def wait_e(k):
            pltpu.make_async_copy(edata.at[0, 0], ed[k], semE[k]).wait()

        def wait_g(k):
            pltpu.make_async_copy(src_tab.at[ed[k].at[0]], rw[k].at[LO],
                                  semG[k]).wait()
            pltpu.make_async_copy(src_tab.at[ed[k].at[1]], rw[k].at[HI],
                                  semG[k]).wait()

        def wait_s(k):
            pltpu.make_async_copy(rw[k].at[LO], acc.at[dx[k].at[0]],
                                  semS[k]).wait()
            pltpu.make_async_copy(rw[k].at[HI], acc.at[dx[k].at[1]],
                                  semS[k]).wait()

        def do_a(st, k, with_s_wait):
            if with_s_wait:
                wait_s(k)
            wait_e(k)
            pltpu.async_copy(src_tab.at[ed[k].at[0]], rw[k].at[LO], semG[k])
            pltpu.async_copy(src_tab.at[ed[k].at[1]], rw[k].at[HI], semG[k])

        def do_b(st, k):
            wait_g(k)
            for g in range(K // 16):
                sl = pl.ds(g * 16, 16)
                dx[k][0, sl] = ed[k][2, sl]
                dx[k][1, sl] = ed[k][3, sl]
            for h in range(2):
                @plsc.parallel_loop(0, K // 16)
                def grp(g):
                    w16 = plsc.bitcast(ed[k][4 + h, pl.ds(g * 16, 16)], F32)
                    for j in range(16):
                        e = h * K + g * 16 + j
                        w = _vsplat(w16, j)
                        rw[k][e, 0:16] = rw[k][e, 0:16] * w
                        rw[k][e, 16:32] = rw[k][e, 16:32] * w
            pltpu.async_copy(rw[k].at[LO], acc.at[dx[k].at[0]], semS[k],
                             add=True)
            pltpu.async_copy(rw[k].at[HI], acc.at[dx[k].at[1]], semS[k],
                             add=True)
            issue_e(st + 3, k)

        issue_e(0, 0)
        issue_e(1, 1)
        issue_e(2, 2)
        do_a(0, 0, False)
        do_a(1, 1, False)
        do_b(0, 0)
        do_a(2, 2, False)
        do_b(1, 1)

        def lbody(j, carry):
            st = 3 * j
            for k in range(3):
                cc = st + k
                do_a(cc, k, True)
                do_b(cc - 1, (k + 2) % 3)
            return carry
        lax.fori_loop(1, NST // 3 + 1, lbody, None)
        wait_s(0)
        wait_s(1)
        wait_g(2)
        wait_e(0)
        wait_e(1)
        plsc.subcore_barrier()

        pltpu.sync_copy(acc.at[pl.ds(s * RPT, RPT)],
                        dst_tab.at[pl.ds(node_off + s * RPT, RPT)])
        pltpu.sync_copy(zeros_slab, acc.at[pl.ds(s * RPT, RPT)])
        plsc.subcore_barrier()

    layer(a0, a1)
    layer(a1, a2)
    layer(a2, a3)

    def accum_tab(tab, rb, idx2, first):
        pltpu.sync_copy(tab.at[idx2], rw0.at[LO])
        rbase = rb * K

        def ad(e, carry):
            eo = rbase + e
            if first:
                rw1[eo, 0:16] = rw0[e, 0:16]
                rw1[eo, 16:32] = rw0[e, 16:32]
            else:
                rw1[eo, 0:16] = rw1[eo, 0:16] + rw0[e, 0:16]
                rw1[eo, 16:32] = rw1[eo, 16:32] + rw0[e, 16:32]
            return carry
        lax.fori_loop(0, K, ad, None)

    for sub in range(2):
        b0 = s * BPT + sub * K
        pltpu.sync_copy(users.at[pl.ds(b0, K)], dx0.at[0])
        pltpu.sync_copy(items.at[pl.ds(b0, K)], dx0.at[1])
        for g in range(K // 16):
            sl = pl.ds(g * 16, 16)
            dx1[0, sl] = dx0[0, sl] + node_off
            dx1[1, sl] = dx0[1, sl] + (node_off + NU)
        for tab, first in ((a0, True), (a1, False), (a2, False), (a3, False)):
            accum_tab(tab, 0, dx1.at[0], first)
            accum_tab(tab, 1, dx1.at[1], first)

        def dot_grp(g, carry):
            riota = jnp.full((16,), g * 16, I32) + lax.iota(I32, 16)

            def dd(d, a):
                cu = plsc.load_gather(rw1, [riota, jnp.full((16,), d, I32)])
                ci = plsc.load_gather(
                    rw1, [riota + K, jnp.full((16,), d, I32)])
                return a + cu * ci
            a = lax.fori_loop(0, H, dd, jnp.zeros((16,), F32))
            part_v[pl.ds(sub * K + g * 16, 16)] = a * (1.0 / 16.0)
            return carry
        lax.fori_loop(0, K // 16, dot_grp, None)

    pltpu.sync_copy(part_v, part.at[pl.ds(c * B + s * BPT, BPT)])


@jax.jit
def kernel(users, items, edge_index, edge_weight, e_user, e_item):
    all_emb = jnp.concatenate(
        [e_user, e_item, jnp.zeros((NP_ - N, D), F32)], axis=0)
    padn = NE_PAD - NE
    src2 = jnp.concatenate(
        [edge_index[0], jnp.zeros((padn,), I32)]).reshape(NST_TOT, 2, K)
    dst2 = jnp.concatenate(
        [edge_index[1], jnp.zeros((padn,), I32)]).reshape(NST_TOT, 2, K)
    w2 = lax.bitcast_convert_type(
        jnp.concatenate([edge_weight, jnp.zeros((padn,), F32)]),
        I32).reshape(NST_TOT, 2, K)
    recs = []
    for cc in range(2):
        r = jnp.concatenate([src2 + cc * NP_, dst2, w2], axis=1)
        recs.append(jnp.concatenate([r, jnp.zeros((3, 6, K), I32)], axis=0))
    edata = jnp.stack(recs, axis=0)
    zeros_slab = jnp.zeros((RPT, H), F32)

    mesh = plsc.VectorSubcoreMesh(core_axis_name="c", subcore_axis_name="s")
    run = pl.kernel(
        _sc_body,
        out_type=[
            jax.ShapeDtypeStruct((2 * B,), F32),
            jax.ShapeDtypeStruct((2 * NP_, H), F32),
            jax.ShapeDtypeStruct((2 * NP_, H), F32),
            jax.ShapeDtypeStruct((2 * NP_, H), F32),
            jax.ShapeDtypeStruct((2 * NP_, H), F32),
        ],
        mesh=mesh,
        compiler_params=pltpu.CompilerParams(
            use_tc_tiling_on_sc=False, needs_layout_passes=False),
        scratch_types=(
            [pltpu.VMEM_SHARED((NP_, H), F32)]
            + [pltpu.VMEM((6, K), I32) for _ in range(3)]
            + [pltpu.VMEM((2, K), I32) for _ in range(3)]
            + [pltpu.VMEM((SE, H), F32) for _ in range(3)]
            + [pltpu.VMEM((BPT,), F32)]
            + [pltpu.SemaphoreType.DMA for _ in range(9)]
        ),
    )
    part = run(all_emb, edata, zeros_slab, users, items)[0]
    return part[:B] + part[B:]
